# Initial kernel scaffold; baseline (speedup 1.0000x reference)
import jax
import jax.numpy as jnp
from jax.experimental import pallas as pl


def kernel(attribute, betas):
    raise NotImplementedError("write your pallas kernel here")


# speedup vs baseline: 35.2184x; 35.2184x over previous
import jax
import jax.numpy as jnp
import numpy as np
from jax import lax
from jax.experimental import pallas as pl
from jax.experimental.pallas import tpu as pltpu

_N_CLS = 1024
_ATT = 512
_LP1 = 16
_SQRT8 = float(np.sqrt(np.float32(8.0)))

_C_BLK = 64
_NBLK = _N_CLS // _C_BLK
_SEEN_BLKS = 768 // _C_BLK


def _body(betas_ref, a_ref, zsl_ref, seen_ref, gz_ref):
    i = pl.program_id(0)
    A = a_ref[...]
    s = jnp.sum(A * A, axis=1)
    recip = 1.0 / jnp.maximum(jnp.sqrt(s), 1e-12)

    P = _C_BLK * _LP1
    p_idx = lax.broadcasted_iota(jnp.int32, (_C_BLK, P), 1)
    c_idx = lax.broadcasted_iota(jnp.int32, (_C_BLK, P), 0)
    W = jnp.where(p_idx == c_idx * _LP1, recip[:, None], 0.0)
    main = lax.dot_general(A, W, (((0,), (0,)), ((), ())),
                           preferred_element_type=jnp.float32)

    a_i = lax.broadcasted_iota(jnp.int32, (_ATT, _LP1), 0)
    j_i = lax.broadcasted_iota(jnp.int32, (_ATT, _LP1), 1)
    vv = jnp.zeros((_ATT, _LP1), jnp.float32)
    for j in range(2, _LP1):
        b = betas_ref[0, j - 2]
        vj = b / jnp.maximum(jnp.abs(b) * _SQRT8, 1e-12)
        vv = jnp.where(j_i == j, vj, vv)
    base = (j_i - 1) * 32
    win = (a_i >= base) & (a_i < base + 8) & (j_i >= 2)
    U = jnp.where(win, vv, 0.0)
    jj = lax.broadcasted_iota(jnp.int32, (_LP1, P), 0)
    pp = lax.broadcasted_iota(jnp.int32, (_LP1, P), 1)
    V = (pp % _LP1 == jj).astype(jnp.float32)
    bg = lax.dot_general(U, V, (((1,), (0,)), ((), ())),
                         preferred_element_type=jnp.float32)

    out = main + bg
    gz_ref[...] = out

    @pl.when(i < _SEEN_BLKS)
    def _():
        seen_ref[...] = out

    @pl.when(i >= _SEEN_BLKS)
    def _():
        zsl_ref[...] = out


def kernel(attribute, betas):
    P = _C_BLK * _LP1
    out_shapes = (
        jax.ShapeDtypeStruct((_ATT, (_N_CLS - 768) * _LP1), jnp.float32),
        jax.ShapeDtypeStruct((_ATT, 768 * _LP1), jnp.float32),
        jax.ShapeDtypeStruct((_ATT, _N_CLS * _LP1), jnp.float32),
    )
    zsl, seen, gz = pl.pallas_call(
        _body,
        grid=(_NBLK,),
        in_specs=[
            pl.BlockSpec(memory_space=pltpu.SMEM),
            pl.BlockSpec((_C_BLK, _ATT), lambda i: (i, 0)),
        ],
        out_specs=[
            pl.BlockSpec((_ATT, P), lambda i: (0, jnp.maximum(i - _SEEN_BLKS, 0))),
            pl.BlockSpec((_ATT, P), lambda i: (0, jnp.minimum(i, _SEEN_BLKS - 1))),
            pl.BlockSpec((_ATT, P), lambda i: (0, i)),
        ],
        out_shape=out_shapes,
    )(betas, attribute)
    return (zsl, seen, gz)
